# Initial kernel scaffold; baseline (speedup 1.0000x reference)
import jax
import jax.numpy as jnp
from jax.experimental import pallas as pl


def kernel(x, codebook):
    raise NotImplementedError("write your pallas kernel here")


# speedup vs baseline: 1.2875x; 1.2875x over previous
import functools

import jax
import jax.numpy as jnp
from jax import lax
from jax.experimental import pallas as pl
from jax.experimental.pallas import tpu as pltpu
from jax.experimental.pallas import tpu_sc as plsc

N_TOK = 16384
DIM = 256
N_EMB = 8192
WINDOWS = ((0, 2736), (2736, 2736), (5472, 2720))
PW = 2816
RB = 512
BIG = 2 ** 30
LOSS_SCALE = 1.25 / float(N_TOK * DIM)


def _dist_argmin_body(xp_ref, cbb_ref, cbf_ref, idx_ref, loss_ref,
                      csq0, csq1, csq2, acc):
    step = pl.program_id(0)
    csqs = (csq0, csq1, csq2)

    @pl.when(step == 0)
    def _init():
        lane = lax.broadcasted_iota(jnp.int32, (1, PW), 1)
        for w, (lo, sz) in enumerate(WINDOWS):
            cf = cbf_ref[w]
            s = jnp.sum(cf * cf, axis=0, keepdims=True)
            csqs[w][...] = jnp.where(lane < sz, s, jnp.float32(jnp.inf))
        acc[...] = jnp.zeros((1, 1), jnp.float32)

    xb = xp_ref[...]
    xsq = jnp.sum(xb * xb, axis=1, keepdims=True)
    xbb = xb.astype(jnp.bfloat16)

    v_run = jnp.full((RB, 1), jnp.inf, jnp.float32)
    r_run = jnp.full((RB, 1), jnp.inf, jnp.float32)
    i_run = jnp.zeros((RB, 1), jnp.int32)
    for w, (lo, sz) in enumerate(WINDOWS):
        s2 = lax.dot_general(xbb, cbb_ref[w], (((1,), (0,)), ((), ())),
                             preferred_element_type=jnp.float32)
        dd = (xsq + csqs[w][...]) - s2
        v = jnp.min(dd, axis=1, keepdims=True)
        iota = lax.broadcasted_iota(jnp.int32, (RB, PW), 1) + lo
        cand = jnp.where(dd == v, iota, BIG)
        i = jnp.min(cand, axis=1, keepdims=True)
        take = v < v_run
        i_run = jnp.where(take, i, i_run)
        r_run = jnp.where(take, v, r_run)
        v_run = jnp.where(take, v, v_run).astype(jnp.bfloat16).astype(jnp.float32)
    idx_ref[...] = i_run
    acc[...] += jnp.sum(r_run).reshape(1, 1)

    @pl.when(step == (N_TOK // RB) - 1)
    def _fin():
        loss_ref[...] = acc[...] * jnp.float32(LOSS_SCALE)


def _dist_argmin(xp, cbb, cbf):
    return pl.pallas_call(
        _dist_argmin_body,
        grid=(N_TOK // RB,),
        in_specs=[pl.BlockSpec((RB, DIM), lambda i: (i, 0)),
                  pl.BlockSpec((3, DIM, PW), lambda i: (0, 0, 0)),
                  pl.BlockSpec((3, DIM, PW), lambda i: (0, 0, 0))],
        out_specs=[pl.BlockSpec((RB, 1), lambda i: (i, 0)),
                   pl.BlockSpec((1, 1), lambda i: (0, 0))],
        out_shape=[jax.ShapeDtypeStruct((N_TOK, 1), jnp.int32),
                   jax.ShapeDtypeStruct((1, 1), jnp.float32)],
        scratch_shapes=[pltpu.VMEM((1, PW), jnp.float32),
                        pltpu.VMEM((1, PW), jnp.float32),
                        pltpu.VMEM((1, PW), jnp.float32),
                        pltpu.VMEM((1, 1), jnp.float32)],
    )(xp, cbb, cbf)


def _codebook_windows(cb):
    cbt2 = (2.0 * cb).astype(jnp.bfloat16).T
    cbtf = cb.T
    bs, fs = [], []
    for lo, sz in WINDOWS:
        bs.append(jnp.pad(cbt2[:, lo:lo + sz], ((0, 0), (0, PW - sz))))
        fs.append(jnp.pad(cbtf[:, lo:lo + sz], ((0, 0), (0, PW - sz))))
    return jnp.stack(bs), jnp.stack(fs)


def _sc_gather(table, idx):
    info = plsc.get_sparse_core_info()
    nc, ns = info.num_cores, info.num_subcores
    nw = nc * ns
    b_per_w = N_TOK // nw
    chunk = 128
    n_chunks = b_per_w // chunk
    mesh = plsc.VectorSubcoreMesh(core_axis_name="c", subcore_axis_name="s")

    @functools.partial(
        pl.kernel, mesh=mesh,
        out_type=jax.ShapeDtypeStruct((N_TOK, DIM), jnp.float32),
        scratch_types=[pltpu.VMEM((chunk,), jnp.int32),
                       pltpu.VMEM((chunk, DIM), jnp.float32),
                       pltpu.SemaphoreType.DMA],
    )
    def k(table_hbm, idx_hbm, out_hbm, idx_v, rows_v, sem):
        wid = lax.axis_index("s") * nc + lax.axis_index("c")
        base = wid * b_per_w
        for c in range(n_chunks):
            off = base + c * chunk
            pltpu.sync_copy(idx_hbm.at[pl.ds(off, chunk)], idx_v)
            pltpu.async_copy(table_hbm.at[idx_v], rows_v, sem).wait()
            pltpu.sync_copy(rows_v, out_hbm.at[pl.ds(off, chunk)])

    return k(table, idx)


def kernel(x, codebook):
    xp = jnp.transpose(x, (0, 2, 3, 1)).reshape(N_TOK, DIM)
    cbb, cbf = _codebook_windows(codebook)
    idx, loss = _dist_argmin(xp, cbb, cbf)
    idx_flat = idx.reshape(N_TOK)
    quantized = _sc_gather(codebook, idx_flat)
    quantized_out = jnp.transpose(
        quantized.reshape(16, 32, 32, DIM), (0, 3, 1, 2))
    vq_loss = loss[0, 0]
    encodings = idx_flat.reshape(16, 32, 32)
    return (quantized_out, vq_loss, encodings)


# speedup vs baseline: 1.5098x; 1.1727x over previous
import functools

import jax
import jax.numpy as jnp
from jax import lax
from jax.experimental import pallas as pl
from jax.experimental.pallas import tpu as pltpu
from jax.experimental.pallas import tpu_sc as plsc

N_TOK = 16384
DIM = 256
N_EMB = 8192
WINDOWS = ((0, 2736), (2736, 2736), (5472, 2720))
PW = 2816
RB = 512
IDX_BASE = 8388608.0
IDX_BIG = 33554432.0
LOSS_SCALE = 1.25 / float(N_TOK * DIM)


def _dist_argmin_body(xp_ref, cb2b_ref, csqw_ref, iotaw_ref,
                      idx_ref, loss_ref, acc):
    step = pl.program_id(0)

    @pl.when(step == 0)
    def _init():
        acc[...] = jnp.zeros((1, 1), jnp.float32)

    xb = xp_ref[...]
    xsq = jnp.sum(xb * xb, axis=1, keepdims=True)
    xbb = xb.astype(jnp.bfloat16)

    v_run = jnp.full((RB, 1), jnp.inf, jnp.float32)
    r_run = jnp.full((RB, 1), jnp.inf, jnp.float32)
    i_run = jnp.full((RB, 1), IDX_BIG, jnp.float32)
    for w, (lo, sz) in enumerate(WINDOWS):
        cbw = cb2b_ref[pl.ds(lo, sz), :]
        s2 = lax.dot_general(xbb, cbw, (((1,), (1,)), ((), ())),
                             preferred_element_type=jnp.float32)
        dd = (xsq + csqw_ref[w, :, :sz]) - s2
        v = jnp.min(dd, axis=1, keepdims=True)
        cand = jnp.where(dd == v, iotaw_ref[w, :, :sz], IDX_BIG)
        i = jnp.min(cand, axis=1, keepdims=True)
        take = v < v_run
        i_run = jnp.where(take, i, i_run)
        r_run = jnp.where(take, v, r_run)
        v_run = jnp.where(take, v, v_run).astype(jnp.bfloat16).astype(jnp.float32)
    idx_ref[...] = i_run.astype(jnp.int32) - jnp.int32(IDX_BASE)
    acc[...] += jnp.sum(r_run).reshape(1, 1)

    @pl.when(step == (N_TOK // RB) - 1)
    def _fin():
        loss_ref[...] = acc[...] * jnp.float32(LOSS_SCALE)


def _dist_argmin(xp, cb2b, csqw, iotaw):
    return pl.pallas_call(
        _dist_argmin_body,
        grid=(N_TOK // RB,),
        in_specs=[pl.BlockSpec((RB, DIM), lambda i: (i, 0)),
                  pl.BlockSpec((N_EMB, DIM), lambda i: (0, 0)),
                  pl.BlockSpec((3, 1, PW), lambda i: (0, 0, 0)),
                  pl.BlockSpec((3, 1, PW), lambda i: (0, 0, 0))],
        out_specs=[pl.BlockSpec((RB, 1), lambda i: (i, 0)),
                   pl.BlockSpec((1, 1), lambda i: (0, 0))],
        out_shape=[jax.ShapeDtypeStruct((N_TOK, 1), jnp.int32),
                   jax.ShapeDtypeStruct((1, 1), jnp.float32)],
        scratch_shapes=[pltpu.VMEM((1, 1), jnp.float32)],
    )(xp, cb2b, csqw, iotaw)


def _aux_rows(codebook):
    csq = jnp.sum(codebook ** 2, axis=1)
    iota = IDX_BASE + jnp.arange(N_EMB, dtype=jnp.float32)
    pad_c = jnp.full((PW,), jnp.inf, jnp.float32)
    pad_i = jnp.full((PW,), IDX_BIG, jnp.float32)
    cs, it = [], []
    for lo, sz in WINDOWS:
        cs.append(lax.dynamic_update_slice(pad_c, csq[lo:lo + sz], (0,)))
        it.append(lax.dynamic_update_slice(pad_i, iota[lo:lo + sz], (0,)))
    return (jnp.stack(cs).reshape(3, 1, PW),
            jnp.stack(it).reshape(3, 1, PW))


def _sc_gather(table, idx):
    info = plsc.get_sparse_core_info()
    nc, ns = info.num_cores, info.num_subcores
    nw = nc * ns
    b_per_w = N_TOK // nw
    chunk = 128
    n_chunks = b_per_w // chunk
    mesh = plsc.VectorSubcoreMesh(core_axis_name="c", subcore_axis_name="s")

    @functools.partial(
        pl.kernel, mesh=mesh,
        out_type=jax.ShapeDtypeStruct((N_TOK, DIM), jnp.float32),
        scratch_types=[pltpu.VMEM((chunk,), jnp.int32),
                       pltpu.VMEM((chunk,), jnp.int32),
                       pltpu.VMEM((chunk, DIM), jnp.float32),
                       pltpu.VMEM((chunk, DIM), jnp.float32),
                       pltpu.SemaphoreType.DMA,
                       pltpu.SemaphoreType.DMA],
    )
    def k(table_hbm, idx_hbm, out_hbm,
          idx_v0, idx_v1, rows_v0, rows_v1, sem0, sem1):
        wid = lax.axis_index("s") * nc + lax.axis_index("c")
        base = wid * b_per_w
        idx_bufs = (idx_v0, idx_v1)
        row_bufs = (rows_v0, rows_v1)
        sems = (sem0, sem1)
        handles = [None, None]
        for c in range(n_chunks):
            b = c % 2
            off = base + c * chunk
            if handles[b] is not None:
                handles[b].wait()
                pltpu.sync_copy(row_bufs[b],
                                out_hbm.at[pl.ds(off - 2 * chunk, chunk)])
            pltpu.sync_copy(idx_hbm.at[pl.ds(off, chunk)], idx_bufs[b])
            handles[b] = pltpu.async_copy(
                table_hbm.at[idx_bufs[b]], row_bufs[b], sems[b])
        for c in range(n_chunks - 2, n_chunks):
            b = c % 2
            off = base + c * chunk
            handles[b].wait()
            pltpu.sync_copy(row_bufs[b], out_hbm.at[pl.ds(off, chunk)])

    return k(table, idx)


def kernel(x, codebook):
    xp = jnp.transpose(x, (0, 2, 3, 1)).reshape(N_TOK, DIM)
    cb2b = (2.0 * codebook).astype(jnp.bfloat16)
    csqw, iotaw = _aux_rows(codebook)
    idx, loss = _dist_argmin(xp, cb2b, csqw, iotaw)
    idx_flat = idx.reshape(N_TOK)
    quantized = _sc_gather(codebook, idx_flat)
    quantized_out = jnp.transpose(
        quantized.reshape(16, 32, 32, DIM), (0, 3, 1, 2))
    vq_loss = loss[0, 0]
    encodings = idx_flat.reshape(16, 32, 32)
    return (quantized_out, vq_loss, encodings)
